# Initial kernel scaffold; baseline (speedup 1.0000x reference)
import jax
import jax.numpy as jnp
from jax.experimental import pallas as pl


def kernel(x, Wr, br, W1, b1, W2, b2):
    raise NotImplementedError("write your pallas kernel here")


# speedup vs baseline: 14.8765x; 14.8765x over previous
import functools

import jax
import jax.numpy as jnp
from jax.experimental import pallas as pl
from jax.experimental.pallas import tpu as pltpu

N = 4096
C = 768
E = 64
H = 3072
M = 128
NB = N // M + E
P = NB * M
RB = 512


def _router_body(x_ref, wr_ref, br_ref, idx_ref, gate_ref):
    logits = jnp.dot(x_ref[...], wr_ref[...],
                     preferred_element_type=jnp.float32) + br_ref[...]
    m = jnp.max(logits, axis=-1, keepdims=True)
    s = jnp.sum(jnp.exp(logits - m), axis=-1, keepdims=True)
    idx_ref[...] = jnp.argmax(logits, axis=-1).astype(jnp.int32)
    gate_ref[...] = (1.0 / s)[:, 0]


def _router(flat_x, Wr, br):
    return pl.pallas_call(
        _router_body,
        grid=(N // RB,),
        in_specs=[
            pl.BlockSpec((RB, C), lambda i: (i, 0)),
            pl.BlockSpec((C, E), lambda i: (0, 0)),
            pl.BlockSpec((E,), lambda i: (0,)),
        ],
        out_specs=[
            pl.BlockSpec((RB,), lambda i: (i,)),
            pl.BlockSpec((RB,), lambda i: (i,)),
        ],
        out_shape=[
            jax.ShapeDtypeStruct((N,), jnp.int32),
            jax.ShapeDtypeStruct((N,), jnp.float32),
        ],
    )(flat_x, Wr, br)


def _expert_body(be_ref, na_ref, x_ref, w1_ref, b1_ref, w2_ref, b2_ref,
                 g_ref, y_ref):
    b = pl.program_id(0)

    @pl.when(b < na_ref[0])
    def _():
        h = jnp.dot(x_ref[...], w1_ref[0],
                    preferred_element_type=jnp.float32) + b1_ref[0]
        h = 0.5 * h * (1.0 + jax.lax.erf(h * 0.7071067811865476))
        y = jnp.dot(h, w2_ref[0],
                    preferred_element_type=jnp.float32) + b2_ref[0]
        y_ref[...] = y * g_ref[...]


def _experts(block_expert, num_active, xg, W1, b1, W2, b2, gates2d):
    grid_spec = pltpu.PrefetchScalarGridSpec(
        num_scalar_prefetch=2,
        grid=(NB,),
        in_specs=[
            pl.BlockSpec((M, C), lambda b, be, na: (b, 0)),
            pl.BlockSpec((1, C, H), lambda b, be, na: (be[b], 0, 0)),
            pl.BlockSpec((1, 1, H), lambda b, be, na: (be[b], 0, 0)),
            pl.BlockSpec((1, H, C), lambda b, be, na: (be[b], 0, 0)),
            pl.BlockSpec((1, 1, C), lambda b, be, na: (be[b], 0, 0)),
            pl.BlockSpec((M, 1), lambda b, be, na: (b, 0)),
        ],
        out_specs=pl.BlockSpec((M, C), lambda b, be, na: (b, 0)),
    )
    return pl.pallas_call(
        _expert_body,
        grid_spec=grid_spec,
        out_shape=jax.ShapeDtypeStruct((P, C), jnp.float32),
        compiler_params=pltpu.CompilerParams(
            dimension_semantics=("arbitrary",),
        ),
    )(block_expert, num_active, xg, W1, b1, W2, b2, gates2d)


def kernel(x, Wr, br, W1, b1, W2, b2):
    Bv, Tv, Cv = x.shape
    flat_x = x.reshape(N, C)

    eidx, gate = _router(flat_x, Wr, br)

    oh = (eidx[:, None] == jnp.arange(E, dtype=jnp.int32)[None, :]
          ).astype(jnp.int32)
    rank = jnp.take_along_axis(jnp.cumsum(oh, axis=0) - oh,
                               eidx[:, None], axis=1)[:, 0]
    counts = jnp.sum(oh, axis=0)
    nb_e = (counts + (M - 1)) // M
    blk_cum = jnp.cumsum(nb_e)
    blk_start = blk_cum - nb_e
    num_active = blk_cum[E - 1:E]
    slot = blk_start[eidx] * M + rank

    src = jnp.zeros((P,), jnp.int32).at[slot].set(
        jnp.arange(N, dtype=jnp.int32))
    gates_p = jnp.zeros((P,), jnp.float32).at[slot].set(gate)
    block_expert = jnp.minimum(
        jnp.searchsorted(blk_cum, jnp.arange(NB, dtype=jnp.int32),
                         side="right").astype(jnp.int32), E - 1)

    xg = jnp.take(flat_x, src, axis=0)

    y = _experts(block_expert, num_active, xg, W1, b1[:, None, :],
                 W2, b2[:, None, :], gates_p[:, None])

    out = jnp.take(y, slot, axis=0)
    return out.reshape(Bv, Tv, Cv)
